# baseline (speedup 1.0000x reference)
import functools

import jax
import jax.numpy as jnp
from jax import lax
from jax.experimental import pallas as pl
from jax.experimental.pallas import tpu as pltpu
from jax.experimental.pallas import tpu_sc as plsc

N_NODES = 1000000
MEM_DIM = 64
MSG_DIM = 128
BATCH = 16384

NC = 2
NS = 16
NW = NC * NS
BPW = BATCH // NW
CH = 128
NCH = BPW // CH
IDX_ROWS = BATCH // CH

PACK_ROWS = N_NODES // 2
T_LANES = 2048
T_ROWS = T_LANES // 2
T_GRID = (N_NODES + T_LANES - 1) // T_LANES


def _wid():
    return lax.axis_index("s") * NC + lax.axis_index("c")


@functools.cache
def _make_sc_gather():
    mesh = plsc.VectorSubcoreMesh(
        core_axis_name="c", subcore_axis_name="s", num_cores=NC, num_subcores=NS
    )

    @functools.partial(
        pl.kernel,
        mesh=mesh,
        out_type=jax.ShapeDtypeStruct((BATCH, MEM_DIM), jnp.float32),
        scratch_types=[
            pltpu.VMEM((NCH, CH), jnp.int32),
            pltpu.VMEM((BPW, MEM_DIM), jnp.float32),
            pltpu.SemaphoreType.DMA,
        ],
        compiler_params=pltpu.CompilerParams(use_tc_tiling_on_sc=False),
    )
    def _sc_gather(table_hbm, idx_hbm, out_hbm, idx_v, rows_v, sem):
        wid = _wid()
        base = wid * NCH
        pltpu.sync_copy(idx_hbm.at[pl.ds(base, NCH)], idx_v)
        cps = [
            pltpu.async_copy(
                table_hbm.at[idx_v.at[j]], rows_v.at[pl.ds(j * CH, CH)], sem
            )
            for j in range(NCH)
        ]
        for cp in cps:
            cp.wait()
        pltpu.sync_copy(rows_v, out_hbm.at[pl.ds(wid * BPW, BPW)])

    return _sc_gather


@functools.cache
def _make_sc_scatter():
    mesh = plsc.VectorSubcoreMesh(
        core_axis_name="c", subcore_axis_name="s", num_cores=NC, num_subcores=NS
    )

    @functools.partial(
        pl.kernel,
        mesh=mesh,
        out_type=jax.ShapeDtypeStruct((8,), jnp.int32),
        scratch_types=[
            pltpu.VMEM((NCH, CH), jnp.int32),
            pltpu.VMEM((NCH, CH), jnp.int32),
            pltpu.VMEM((CH, MEM_DIM), jnp.float32),
            pltpu.VMEM((CH,), jnp.float32),
            pltpu.SemaphoreType.DMA,
        ],
        compiler_params=pltpu.CompilerParams(use_tc_tiling_on_sc=False),
    )
    def _sc_scatter(nid_hbm, w_hbm, upd_hbm, lu_vals_hbm, mem_ref, lu_ref,
                    dummy_out, nid_v, w_v, rows_v, luv_v, sem):
        wid = _wid()
        base = wid * NCH
        pltpu.sync_copy(nid_hbm.at[pl.ds(base, NCH)], nid_v)
        pltpu.sync_copy(w_hbm.at[pl.ds(base, NCH)], w_v)
        pltpu.sync_copy(lu_vals_hbm, luv_v)
        for j in range(NCH):
            pltpu.async_copy(upd_hbm.at[w_v.at[j]], rows_v, sem).wait()
            pltpu.async_copy(rows_v, mem_ref.at[nid_v.at[j]], sem).wait()
            pltpu.async_copy(luv_v, lu_ref.at[nid_v.at[j]], sem).wait()

    return _sc_scatter


def _detile_body(in_ref, out_ref):
    t = in_ref[...].T.reshape(T_ROWS, 2, MEM_DIM)
    out_ref[:, 0:MEM_DIM] = t[:, 0, :]
    out_ref[:, MEM_DIM:] = t[:, 1, :]


def _retile_body(in_ref, out_ref):
    x = in_ref[...]
    x3 = jnp.stack([x[:, 0:MEM_DIM], x[:, MEM_DIM:]], axis=1)
    out_ref[...] = x3.reshape(T_LANES, MEM_DIM).T


def _detile(mem_t):
    return pl.pallas_call(
        _detile_body,
        grid=(T_GRID,),
        in_specs=[pl.BlockSpec((MEM_DIM, T_LANES), lambda i: (0, i))],
        out_specs=pl.BlockSpec((T_ROWS, 128), lambda i: (i, 0)),
        out_shape=jax.ShapeDtypeStruct((PACK_ROWS, 128), jnp.float32),
    )(mem_t)


def _retile(packed):
    return pl.pallas_call(
        _retile_body,
        grid=(T_GRID,),
        in_specs=[pl.BlockSpec((T_ROWS, 128), lambda i: (i, 0))],
        out_specs=pl.BlockSpec((MEM_DIM, T_LANES), lambda i: (0, i)),
        out_shape=jax.ShapeDtypeStruct((MEM_DIM, N_NODES), jnp.float32),
    )(packed)


def _gru_body(msg_ref, h_ref, wr, wz, wn, ur, uz, un, brz, bni, bnh, out_ref):
    msg = msg_ref[...]
    h = h_ref[...]
    gi_r = jnp.dot(msg, wr[...], preferred_element_type=jnp.float32)
    gi_z = jnp.dot(msg, wz[...], preferred_element_type=jnp.float32)
    gi_n = jnp.dot(msg, wn[...], preferred_element_type=jnp.float32)
    gh_r = jnp.dot(h, ur[...], preferred_element_type=jnp.float32)
    gh_z = jnp.dot(h, uz[...], preferred_element_type=jnp.float32)
    gh_n = jnp.dot(h, un[...], preferred_element_type=jnp.float32)
    r = jax.nn.sigmoid(gi_r + gh_r + brz[0:1, 0:MEM_DIM])
    z = jax.nn.sigmoid(gi_z + gh_z + brz[0:1, MEM_DIM:2 * MEM_DIM])
    n = jnp.tanh(gi_n + bni[...] + r * (gh_n + bnh[...]))
    out_ref[...] = (1.0 - z) * n + z * h


_GRU_BB = 2048


def _tc_gru(msg, h, wr, wz, wn, ur, uz, un, brz, bni, bnh):
    grid = (BATCH // _GRU_BB,)
    full = lambda i: (0, 0)
    return pl.pallas_call(
        _gru_body,
        grid=grid,
        in_specs=[
            pl.BlockSpec((_GRU_BB, MSG_DIM), lambda i: (i, 0)),
            pl.BlockSpec((_GRU_BB, MEM_DIM), lambda i: (i, 0)),
            pl.BlockSpec((MSG_DIM, MEM_DIM), full),
            pl.BlockSpec((MSG_DIM, MEM_DIM), full),
            pl.BlockSpec((MSG_DIM, MEM_DIM), full),
            pl.BlockSpec((MEM_DIM, MEM_DIM), full),
            pl.BlockSpec((MEM_DIM, MEM_DIM), full),
            pl.BlockSpec((MEM_DIM, MEM_DIM), full),
            pl.BlockSpec((1, 2 * MEM_DIM), full),
            pl.BlockSpec((1, MEM_DIM), full),
            pl.BlockSpec((1, MEM_DIM), full),
        ],
        out_specs=pl.BlockSpec((_GRU_BB, MEM_DIM), lambda i: (i, 0)),
        out_shape=jax.ShapeDtypeStruct((BATCH, MEM_DIM), jnp.float32),
    )(msg, h, wr, wz, wn, ur, uz, un, brz, bni, bnh)


def kernel(memory, last_update, unique_nids, unique_msg, time, W_ih, W_hh, b_ih, b_hh):
    nids = jnp.asarray(unique_nids, jnp.int32)
    idx2d = nids.reshape(IDX_ROWS, CH)

    packed = _detile(memory.T)
    mem_lin = packed.reshape(N_NODES, MEM_DIM)

    mem_ref = jax.new_ref(mem_lin)
    lu_ref = jax.new_ref(last_update)

    h = _make_sc_gather()(mem_ref, idx2d)

    wr = W_ih[0:MEM_DIM].T
    wz = W_ih[MEM_DIM:2 * MEM_DIM].T
    wn = W_ih[2 * MEM_DIM:].T
    ur = W_hh[0:MEM_DIM].T
    uz = W_hh[MEM_DIM:2 * MEM_DIM].T
    un = W_hh[2 * MEM_DIM:].T
    brz = (b_ih[0:2 * MEM_DIM] + b_hh[0:2 * MEM_DIM]).reshape(1, 2 * MEM_DIM)
    bni = b_ih[2 * MEM_DIM:].reshape(1, MEM_DIM)
    bnh = b_hh[2 * MEM_DIM:].reshape(1, MEM_DIM)

    upd = _tc_gru(unique_msg, h, wr, wz, wn, ur, uz, un, brz, bni, bnh)

    arange = jnp.arange(BATCH, dtype=jnp.int32)
    pos = jnp.zeros((N_NODES,), jnp.int32).at[nids].set(arange)
    w2d = pos[nids].reshape(IDX_ROWS, CH)

    lu_vals = jnp.full((CH,), time, dtype=jnp.float32)

    _make_sc_scatter()(idx2d, w2d, upd, lu_vals, mem_ref, lu_ref)

    out_lin = jax.freeze(mem_ref)
    new_memory = _retile(out_lin.reshape(PACK_ROWS, 128)).T
    return new_memory, jax.freeze(lu_ref)


# speedup vs baseline: 2.9936x; 2.9936x over previous
import functools

import jax
import jax.numpy as jnp
from jax import lax
from jax.experimental import pallas as pl
from jax.experimental.pallas import tpu as pltpu
from jax.experimental.pallas import tpu_sc as plsc

N_NODES = 1000000
MEM_DIM = 64
MSG_DIM = 128
BATCH = 16384

NC = 2
NS = 16
NW = NC * NS
BPW = BATCH // NW
CH = 128
NCH = BPW // CH
IDX_ROWS = BATCH // CH

TB = 8192
T_GRID = (N_NODES + 2 * TB - 1) // (2 * TB)
PACK_ROWS = T_GRID * TB
N_LIN = 2 * PACK_ROWS


def _wid():
    return lax.axis_index("s") * NC + lax.axis_index("c")


@functools.cache
def _make_sc_gather():
    mesh = plsc.VectorSubcoreMesh(
        core_axis_name="c", subcore_axis_name="s", num_cores=NC, num_subcores=NS
    )

    @functools.partial(
        pl.kernel,
        mesh=mesh,
        out_type=jax.ShapeDtypeStruct((BATCH, MEM_DIM), jnp.float32),
        scratch_types=[
            pltpu.VMEM((NCH, CH), jnp.int32),
            pltpu.VMEM((BPW, MEM_DIM), jnp.float32),
            pltpu.SemaphoreType.DMA,
        ],
        compiler_params=pltpu.CompilerParams(use_tc_tiling_on_sc=False),
    )
    def _sc_gather(table_hbm, idx_hbm, out_hbm, idx_v, rows_v, sem):
        wid = _wid()
        base = wid * NCH
        pltpu.sync_copy(idx_hbm.at[pl.ds(base, NCH)], idx_v)
        cps = [
            pltpu.async_copy(
                table_hbm.at[idx_v.at[j]], rows_v.at[pl.ds(j * CH, CH)], sem
            )
            for j in range(NCH)
        ]
        for cp in cps:
            cp.wait()
        pltpu.sync_copy(rows_v, out_hbm.at[pl.ds(wid * BPW, BPW)])

    return _sc_gather


@functools.cache
def _make_sc_scatter():
    mesh = plsc.VectorSubcoreMesh(
        core_axis_name="c", subcore_axis_name="s", num_cores=NC, num_subcores=NS
    )

    @functools.partial(
        pl.kernel,
        mesh=mesh,
        out_type=jax.ShapeDtypeStruct((8,), jnp.int32),
        scratch_types=[
            pltpu.VMEM((NCH, CH), jnp.int32),
            pltpu.VMEM((NCH, CH), jnp.int32),
            pltpu.VMEM((NCH, CH), jnp.int32),
            pltpu.VMEM((CH, MEM_DIM), jnp.float32),
            pltpu.VMEM((CH,), jnp.float32),
            pltpu.SemaphoreType.DMA,
        ],
        compiler_params=pltpu.CompilerParams(use_tc_tiling_on_sc=False),
    )
    def _sc_scatter(sig_hbm, nid_hbm, w_hbm, upd_hbm, lu_vals_hbm, mem_ref,
                    lu_ref, dummy_out, sig_v, nid_v, w_v, rows_v, luv_v, sem):
        wid = _wid()
        base = wid * NCH
        pltpu.sync_copy(sig_hbm.at[pl.ds(base, NCH)], sig_v)
        pltpu.sync_copy(nid_hbm.at[pl.ds(base, NCH)], nid_v)
        pltpu.sync_copy(w_hbm.at[pl.ds(base, NCH)], w_v)
        pltpu.sync_copy(lu_vals_hbm, luv_v)
        for j in range(NCH):
            pltpu.async_copy(upd_hbm.at[w_v.at[j]], rows_v, sem).wait()
            pltpu.async_copy(rows_v, mem_ref.at[sig_v.at[j]], sem).wait()
            pltpu.async_copy(luv_v, lu_ref.at[nid_v.at[j]], sem).wait()

    return _sc_scatter


def _detile_body(in_ref, eye_ref, out_ref):
    x = in_ref[...]
    x2 = jnp.concatenate([x[:, 0:TB], x[:, TB:2 * TB]], axis=0)
    out_ref[...] = lax.dot_general(
        x2, eye_ref[...], (((0,), (0,)), ((), ())),
        preferred_element_type=jnp.float32)


def _retile_body(in_ref, eye_ref, out_ref):
    y = lax.dot_general(
        eye_ref[...], in_ref[...], (((1,), (1,)), ((), ())),
        preferred_element_type=jnp.float32)
    out_ref[...] = jnp.concatenate([y[0:MEM_DIM, :], y[MEM_DIM:, :]], axis=1)


def _detile(mem_t, eye128):
    return pl.pallas_call(
        _detile_body,
        grid=(T_GRID,),
        in_specs=[
            pl.BlockSpec((MEM_DIM, 2 * TB), lambda i: (0, i)),
            pl.BlockSpec((128, 128), lambda i: (0, 0)),
        ],
        out_specs=pl.BlockSpec((TB, 128), lambda i: (i, 0)),
        out_shape=jax.ShapeDtypeStruct((PACK_ROWS, 128), jnp.float32),
    )(mem_t, eye128)


def _retile(packed, eye128):
    return pl.pallas_call(
        _retile_body,
        grid=(T_GRID,),
        in_specs=[
            pl.BlockSpec((TB, 128), lambda i: (i, 0)),
            pl.BlockSpec((128, 128), lambda i: (0, 0)),
        ],
        out_specs=pl.BlockSpec((MEM_DIM, 2 * TB), lambda i: (0, i)),
        out_shape=jax.ShapeDtypeStruct((MEM_DIM, N_NODES), jnp.float32),
    )(packed, eye128)


def _gru_body(msg_ref, h_ref, wr, wz, wn, ur, uz, un, brz, bni, bnh, out_ref):
    msg = msg_ref[...]
    h = h_ref[...]
    gi_r = jnp.dot(msg, wr[...], preferred_element_type=jnp.float32)
    gi_z = jnp.dot(msg, wz[...], preferred_element_type=jnp.float32)
    gi_n = jnp.dot(msg, wn[...], preferred_element_type=jnp.float32)
    gh_r = jnp.dot(h, ur[...], preferred_element_type=jnp.float32)
    gh_z = jnp.dot(h, uz[...], preferred_element_type=jnp.float32)
    gh_n = jnp.dot(h, un[...], preferred_element_type=jnp.float32)
    r = jax.nn.sigmoid(gi_r + gh_r + brz[0:1, 0:MEM_DIM])
    z = jax.nn.sigmoid(gi_z + gh_z + brz[0:1, MEM_DIM:2 * MEM_DIM])
    n = jnp.tanh(gi_n + bni[...] + r * (gh_n + bnh[...]))
    out_ref[...] = (1.0 - z) * n + z * h


_GRU_BB = 2048


def _tc_gru(msg, h, wr, wz, wn, ur, uz, un, brz, bni, bnh):
    grid = (BATCH // _GRU_BB,)
    full = lambda i: (0, 0)
    return pl.pallas_call(
        _gru_body,
        grid=grid,
        in_specs=[
            pl.BlockSpec((_GRU_BB, MSG_DIM), lambda i: (i, 0)),
            pl.BlockSpec((_GRU_BB, MEM_DIM), lambda i: (i, 0)),
            pl.BlockSpec((MSG_DIM, MEM_DIM), full),
            pl.BlockSpec((MSG_DIM, MEM_DIM), full),
            pl.BlockSpec((MSG_DIM, MEM_DIM), full),
            pl.BlockSpec((MEM_DIM, MEM_DIM), full),
            pl.BlockSpec((MEM_DIM, MEM_DIM), full),
            pl.BlockSpec((MEM_DIM, MEM_DIM), full),
            pl.BlockSpec((1, 2 * MEM_DIM), full),
            pl.BlockSpec((1, MEM_DIM), full),
            pl.BlockSpec((1, MEM_DIM), full),
        ],
        out_specs=pl.BlockSpec((_GRU_BB, MEM_DIM), lambda i: (i, 0)),
        out_shape=jax.ShapeDtypeStruct((BATCH, MEM_DIM), jnp.float32),
    )(msg, h, wr, wz, wn, ur, uz, un, brz, bni, bnh)


def kernel(memory, last_update, unique_nids, unique_msg, time, W_ih, W_hh, b_ih, b_hh):
    nids = jnp.asarray(unique_nids, jnp.int32)
    idx2d = nids.reshape(IDX_ROWS, CH)
    slab = nids // TB
    sig = 2 * ((slab >> 1) * TB + (nids % TB)) + (slab & 1)
    sig2d = sig.reshape(IDX_ROWS, CH)

    eye128 = jnp.eye(128, dtype=jnp.float32)

    packed = _detile(memory.T, eye128)
    mem_lin = packed.reshape(N_LIN, MEM_DIM)

    mem_ref = jax.new_ref(mem_lin)
    lu_ref = jax.new_ref(last_update)

    h = _make_sc_gather()(mem_ref, sig2d)

    wr = W_ih[0:MEM_DIM].T
    wz = W_ih[MEM_DIM:2 * MEM_DIM].T
    wn = W_ih[2 * MEM_DIM:].T
    ur = W_hh[0:MEM_DIM].T
    uz = W_hh[MEM_DIM:2 * MEM_DIM].T
    un = W_hh[2 * MEM_DIM:].T
    brz = (b_ih[0:2 * MEM_DIM] + b_hh[0:2 * MEM_DIM]).reshape(1, 2 * MEM_DIM)
    bni = b_ih[2 * MEM_DIM:].reshape(1, MEM_DIM)
    bnh = b_hh[2 * MEM_DIM:].reshape(1, MEM_DIM)

    upd = _tc_gru(unique_msg, h, wr, wz, wn, ur, uz, un, brz, bni, bnh)

    arange = jnp.arange(BATCH, dtype=jnp.int32)
    pos = jnp.zeros((N_NODES,), jnp.int32).at[nids].set(arange)
    w2d = pos[nids].reshape(IDX_ROWS, CH)

    lu_vals = jnp.full((CH,), time, dtype=jnp.float32)

    _make_sc_scatter()(sig2d, idx2d, w2d, upd, lu_vals, mem_ref, lu_ref)

    out_lin = jax.freeze(mem_ref)
    new_memory = _retile(out_lin.reshape(PACK_ROWS, 128), eye128).T
    return new_memory, jax.freeze(lu_ref)
